# initial kernel scaffold (unmeasured)
import jax
import jax.numpy as jnp
from jax import lax
from jax.experimental import pallas as pl
from jax.experimental.pallas import tpu as pltpu

N_DEV = 32
BLK = 128
K = 4096
N_OUT = 2048


def kernel(x, w_mat):
    def body(x_ref, w_hbm_ref, out_ref, xbf_ref, xg_ref, wv_ref,
             send_sems, recv_sems, w_sem):
        me = lax.axis_index("i")

        w_copy = pltpu.make_async_copy(w_hbm_ref, wv_ref, w_sem)
        w_copy.start()

        xbf_ref[...] = x_ref[...].astype(jnp.bfloat16)

        barrier = pltpu.get_barrier_semaphore()
        for d in range(1, N_DEV):
            peer = lax.rem(me + d, N_DEV)
            pl.semaphore_signal(
                barrier, inc=1,
                device_id=(peer,), device_id_type=pl.DeviceIdType.MESH,
            )
        pl.semaphore_wait(barrier, N_DEV - 1)

        sends = []
        for d in range(N_DEV):
            j = lax.rem(me + d, N_DEV)
            rdma = pltpu.make_async_remote_copy(
                src_ref=xbf_ref.at[pl.ds(j * BLK, BLK), :],
                dst_ref=xg_ref.at[:, pl.ds(me * BLK, BLK)],
                send_sem=send_sems.at[j],
                recv_sem=recv_sems.at[me],
                device_id=(j,),
                device_id_type=pl.DeviceIdType.MESH,
            )
            rdma.start()
            sends.append(rdma)

        for d in range(N_DEV):
            s = lax.rem(me + d, N_DEV)
            recv = pltpu.make_async_remote_copy(
                src_ref=xbf_ref.at[pl.ds(0, BLK), :],
                dst_ref=xg_ref.at[:, pl.ds(s * BLK, BLK)],
                send_sem=send_sems.at[s],
                recv_sem=recv_sems.at[s],
                device_id=(me,),
                device_id_type=pl.DeviceIdType.MESH,
            )
            recv.wait_recv()

        for rdma in sends:
            rdma.wait_send()

        w_copy.wait()
        y = jnp.dot(
            xg_ref[...].astype(jnp.float32), wv_ref[...],
            preferred_element_type=jnp.float32,
        )
        out_ref[...] = jnp.maximum(y, 0.0)

    return pl.pallas_call(
        body,
        out_shape=jax.ShapeDtypeStruct((BLK, N_OUT), jnp.float32),
        in_specs=[
            pl.BlockSpec(memory_space=pltpu.VMEM),
            pl.BlockSpec(memory_space=pltpu.ANY),
        ],
        out_specs=pl.BlockSpec(memory_space=pltpu.VMEM),
        scratch_shapes=[
            pltpu.VMEM((K, BLK), jnp.bfloat16),
            pltpu.VMEM((BLK, K), jnp.bfloat16),
            pltpu.VMEM((K, N_OUT), jnp.float32),
            pltpu.SemaphoreType.DMA((N_DEV,)),
            pltpu.SemaphoreType.DMA((N_DEV,)),
            pltpu.SemaphoreType.DMA,
        ],
        compiler_params=pltpu.CompilerParams(collective_id=0),
    )(x, w_mat)


# baseline (device time: 28597 ns/iter reference)
import jax
import jax.numpy as jnp
from jax import lax
from jax.experimental import pallas as pl
from jax.experimental.pallas import tpu as pltpu

N_DEV = 32
BLK = 128
K = 4096
N_OUT = 2048


def kernel(x, w_mat):
    def body(x_ref, w_hbm_ref, out_ref, xbf_ref, xg_ref, wv_ref,
             send_sems, recv_sems, w_sem):
        me = lax.axis_index("i")

        w_copy = pltpu.make_async_copy(w_hbm_ref, wv_ref, w_sem)
        w_copy.start()

        xbf_ref[...] = x_ref[...].astype(jnp.bfloat16)

        barrier = pltpu.get_barrier_semaphore()
        for d in range(1, N_DEV):
            peer = lax.rem(me + d, N_DEV)
            pl.semaphore_signal(
                barrier, inc=1,
                device_id=(peer,), device_id_type=pl.DeviceIdType.MESH,
            )
        pl.semaphore_wait(barrier, N_DEV - 1)

        sends = []
        for d in range(N_DEV):
            j = lax.rem(me + d, N_DEV)
            rdma = pltpu.make_async_remote_copy(
                src_ref=xbf_ref.at[pl.ds(j * BLK, BLK), :],
                dst_ref=xg_ref.at[:, pl.ds(me * BLK, BLK)],
                send_sem=send_sems.at[j],
                recv_sem=recv_sems.at[me],
                device_id=(j,),
                device_id_type=pl.DeviceIdType.MESH,
            )
            rdma.start()
            sends.append(rdma)

        for d in range(N_DEV):
            s = lax.rem(me + d, N_DEV)
            recv = pltpu.make_async_remote_copy(
                src_ref=xbf_ref.at[pl.ds(0, BLK), :],
                dst_ref=xg_ref.at[:, pl.ds(s * BLK, BLK)],
                send_sem=send_sems.at[s],
                recv_sem=recv_sems.at[s],
                device_id=(me,),
                device_id_type=pl.DeviceIdType.MESH,
            )
            recv.wait_recv()

        for rdma in sends:
            rdma.wait_send()

        w_copy.wait()
        y = jnp.dot(
            xg_ref[...].astype(jnp.float32), wv_ref[...],
            preferred_element_type=jnp.float32,
        )
        out_ref[...] = jnp.maximum(y, 0.0)

    return pl.pallas_call(
        body,
        out_shape=jax.ShapeDtypeStruct((BLK, N_OUT), jnp.float32),
        in_specs=[
            pl.BlockSpec(memory_space=pltpu.VMEM),
            pl.BlockSpec(memory_space=pl.ANY),
        ],
        out_specs=pl.BlockSpec(memory_space=pltpu.VMEM),
        scratch_shapes=[
            pltpu.VMEM((K, BLK), jnp.bfloat16),
            pltpu.VMEM((BLK, K), jnp.bfloat16),
            pltpu.VMEM((K, N_OUT), jnp.float32),
            pltpu.SemaphoreType.DMA((N_DEV,)),
            pltpu.SemaphoreType.DMA((N_DEV,)),
            pltpu.SemaphoreType.DMA,
        ],
        compiler_params=pltpu.CompilerParams(
            collective_id=0, vmem_limit_bytes=64 * 1024 * 1024,
        ),
    )(x, w_mat)
